# baseline (speedup 1.0000x reference)
import jax
import jax.numpy as jnp
from jax import lax
from jax.experimental import pallas as pl
from jax.experimental.pallas import tpu as pltpu
from jax.experimental.pallas import tpu_sc as plsc

N_NODES = 10000
D_FEAT = 128
N_EDGES = 320000

NC = 2
NS = 16
NW = NC * NS
DH = D_FEAT // NC
CHUNK = 256
N_PAD = 10240
ROWS_PER_TILE = N_PAD // NS
G_ROWS_PER_TILE = 624
NCH = 80
E_PAD = NS * NCH * CHUNK
DCHUNK = 128
DNCH = E_PAD // (NW * DCHUNK)
NBUF = 2

_SC_MESH = dict(core_axis_name="c", subcore_axis_name="s")


def _deg_body(dst_hbm, out_hbm, didx_v, ones_v, zero_v, acc, sem):
    c = lax.axis_index("c")
    s = lax.axis_index("s")
    z16 = jnp.zeros((16,), jnp.float32)
    o16 = jnp.ones((16,), jnp.float32)

    def init_bufs(i, _):
        zero_v[pl.ds(i * 16, 16)] = z16
        return 0

    lax.fori_loop(0, ROWS_PER_TILE // 16, init_bufs, 0)
    for k in range(DCHUNK // 16):
        ones_v[pl.ds(k * 16, 16)] = o16
    pltpu.sync_copy(dst_hbm.at[c, s], didx_v)
    pltpu.sync_copy(zero_v, acc.at[pl.ds(s * ROWS_PER_TILE, ROWS_PER_TILE)])
    plsc.subcore_barrier()

    def fire(j, _):
        pltpu.async_copy(ones_v, acc.at[didx_v.at[j]], sem, add=True)
        return 0

    lax.fori_loop(0, DNCH, fire, 0)

    def drain(j, _):
        pltpu.make_async_copy(ones_v, acc.at[didx_v.at[0]], sem).wait()
        return 0

    lax.fori_loop(0, DNCH, drain, 0)
    plsc.subcore_barrier()
    pltpu.sync_copy(acc.at[pl.ds(s * ROWS_PER_TILE, ROWS_PER_TILE)],
                    out_hbm.at[c, pl.ds(s * ROWS_PER_TILE, ROWS_PER_TILE)])


_deg_call = pl.kernel(
    _deg_body,
    out_type=jax.ShapeDtypeStruct((NC, N_PAD), jnp.float32),
    mesh=plsc.VectorSubcoreMesh(**_SC_MESH),
    scratch_types=[
        pltpu.VMEM((DNCH, DCHUNK), jnp.int32),
        pltpu.VMEM((DCHUNK,), jnp.float32),
        pltpu.VMEM((ROWS_PER_TILE,), jnp.float32),
        pltpu.VMEM_SHARED((N_PAD,), jnp.float32),
        pltpu.SemaphoreType.DMA,
    ],
)


def _scat_body(g0_hbm, g1_hbm, src_hbm, dst_hbm, out_hbm, sidx_v, didx_v,
               rows_v, gsp, acc, isem, dsem, gsem):
    c = lax.axis_index("c")
    s = lax.axis_index("s")
    z16 = jnp.zeros((16,), jnp.float32)

    @pl.when(c == 0)
    def _():
        pltpu.sync_copy(
            g0_hbm.at[pl.ds(s * G_ROWS_PER_TILE, G_ROWS_PER_TILE)],
            gsp.at[pl.ds(s * G_ROWS_PER_TILE, G_ROWS_PER_TILE)])

        @pl.when(s == 0)
        def _():
            tail = NS * G_ROWS_PER_TILE
            pltpu.sync_copy(g0_hbm.at[pl.ds(tail, N_NODES - tail)],
                            gsp.at[pl.ds(tail, N_NODES - tail)])

    @pl.when(c == 1)
    def _():
        pltpu.sync_copy(
            g1_hbm.at[pl.ds(s * G_ROWS_PER_TILE, G_ROWS_PER_TILE)],
            gsp.at[pl.ds(s * G_ROWS_PER_TILE, G_ROWS_PER_TILE)])

        @pl.when(s == 0)
        def _():
            tail = NS * G_ROWS_PER_TILE
            pltpu.sync_copy(g1_hbm.at[pl.ds(tail, N_NODES - tail)],
                            gsp.at[pl.ds(tail, N_NODES - tail)])

    def zero_rows(i, _):
        for k in range(DH // 16):
            rows_v[0, i, pl.ds(k * 16, 16)] = z16
        return 0

    lax.fori_loop(0, CHUNK, zero_rows, 0)
    off = 0
    while off < ROWS_PER_TILE:
        step_rows = min(CHUNK, ROWS_PER_TILE - off)
        pltpu.sync_copy(
            rows_v.at[0, pl.ds(0, step_rows)],
            acc.at[pl.ds(s * ROWS_PER_TILE + off, step_rows)])
        off += step_rows
    pltpu.async_copy(src_hbm.at[s, 0], sidx_v.at[0], isem)
    pltpu.async_copy(src_hbm.at[s, 1], sidx_v.at[1], isem)
    pltpu.async_copy(dst_hbm.at[s, 0], didx_v.at[0], dsem)
    pltpu.async_copy(dst_hbm.at[s, 1], didx_v.at[1], dsem)
    plsc.subcore_barrier()
    pltpu.make_async_copy(src_hbm.at[0, 0], sidx_v.at[0], isem).wait()
    pltpu.async_copy(gsp.at[sidx_v.at[0]], rows_v.at[0], gsem)

    def step(jg, _):
        for b in range(NBUF):
            j = jg * NBUF + b
            rows_b = rows_v.at[b]
            nb = (b + 1) % NBUF

            pltpu.make_async_copy(gsp.at[sidx_v.at[b]], rows_b, gsem).wait()

            @pl.when(j + 1 < NCH)
            def _():
                pltpu.make_async_copy(
                    src_hbm.at[0, 0], sidx_v.at[nb], isem).wait()
                pltpu.async_copy(gsp.at[sidx_v.at[nb]], rows_v.at[nb], gsem)

            @pl.when(j + NBUF < NCH)
            def _():
                pltpu.async_copy(src_hbm.at[s, j + NBUF], sidx_v.at[b], isem)

            pltpu.make_async_copy(dst_hbm.at[0, 0], didx_v.at[b], dsem).wait()
            pltpu.sync_copy(rows_b, acc.at[didx_v.at[b]], add=True)

            @pl.when(j + NBUF < NCH)
            def _():
                pltpu.async_copy(dst_hbm.at[s, j + NBUF], didx_v.at[b], dsem)

        return 0

    lax.fori_loop(0, NCH // NBUF, step, 0)
    plsc.subcore_barrier()
    pltpu.sync_copy(acc.at[pl.ds(s * ROWS_PER_TILE, ROWS_PER_TILE)],
                    out_hbm.at[c, pl.ds(s * ROWS_PER_TILE, ROWS_PER_TILE)])


_scat_call = pl.kernel(
    _scat_body,
    out_type=jax.ShapeDtypeStruct((NC, N_PAD, DH), jnp.float32),
    mesh=plsc.VectorSubcoreMesh(**_SC_MESH),
    compiler_params=pltpu.CompilerParams(use_tc_tiling_on_sc=False),
    scratch_types=[
        pltpu.VMEM((NBUF, CHUNK), jnp.int32),
        pltpu.VMEM((NBUF, CHUNK), jnp.int32),
        pltpu.VMEM((NBUF, CHUNK, DH), jnp.float32),
        pltpu.VMEM_SHARED((N_NODES, DH), jnp.float32),
        pltpu.VMEM_SHARED((N_PAD, DH), jnp.float32),
        pltpu.SemaphoreType.DMA,
        pltpu.SemaphoreType.DMA,
        pltpu.SemaphoreType.DMA,
    ],
)


_R = 1024


def _mm_scale_body(x_ref, w_ref, deg_ref, g_ref):
    h = jnp.dot(x_ref[...], w_ref[...], preferred_element_type=jnp.float32)
    deg = deg_ref[0, :] + deg_ref[1, :] + 1.0
    dinv = lax.rsqrt(deg)
    g = h * dinv[:, None]
    g_ref[0] = g[:, :DH]
    g_ref[1] = g[:, DH:]


def _mm_scale(x, W, deg01):
    return pl.pallas_call(
        _mm_scale_body,
        grid=(N_PAD // _R,),
        in_specs=[
            pl.BlockSpec((_R, D_FEAT), lambda i: (i, 0)),
            pl.BlockSpec((D_FEAT, D_FEAT), lambda i: (0, 0)),
            pl.BlockSpec((NC, _R), lambda i: (0, i)),
        ],
        out_specs=pl.BlockSpec((NC, _R, DH), lambda i: (0, i, 0)),
        out_shape=jax.ShapeDtypeStruct((NC, N_NODES, DH), jnp.float32),
    )(x, W, deg01)


def _epi_body(scat_ref, g_ref, deg_ref, b_ref, w2_ref, b2_ref, out_ref):
    deg = deg_ref[0, :] + deg_ref[1, :] + 1.0
    dinv = lax.rsqrt(deg)
    scat = jnp.concatenate([scat_ref[0], scat_ref[1]], axis=1)
    g = jnp.concatenate([g_ref[0], g_ref[1]], axis=1)
    agg = dinv[:, None] * (scat + g)
    z = jax.nn.relu(agg + b_ref[...])
    logits = jnp.sum(z * w2_ref[...], axis=1, keepdims=True) + b2_ref[...]
    out_ref[...] = jax.nn.sigmoid(logits)


def _epilogue(scat, g, deg01, b, W2, b2):
    return pl.pallas_call(
        _epi_body,
        grid=(N_PAD // _R,),
        in_specs=[
            pl.BlockSpec((NC, _R, DH), lambda i: (0, i, 0)),
            pl.BlockSpec((NC, _R, DH), lambda i: (0, i, 0)),
            pl.BlockSpec((NC, _R), lambda i: (0, i)),
            pl.BlockSpec((1, D_FEAT), lambda i: (0, 0)),
            pl.BlockSpec((1, D_FEAT), lambda i: (0, 0)),
            pl.BlockSpec((1, 1), lambda i: (0, 0)),
        ],
        out_specs=pl.BlockSpec((_R, 1), lambda i: (i, 0)),
        out_shape=jax.ShapeDtypeStruct((N_NODES, 1), jnp.float32),
    )(scat, g, deg01, b, W2, b2)


def kernel(x, edge_index, W, b, W2, b2):
    src = edge_index[0].astype(jnp.int32)
    dst = edge_index[1].astype(jnp.int32)
    npad = E_PAD - N_EDGES
    pad_src = jnp.arange(npad, dtype=jnp.int32) % N_NODES
    src_p = jnp.concatenate([src, pad_src])
    dump = N_NODES + jnp.arange(npad, dtype=jnp.int32) % (N_PAD - N_NODES)
    dst_p = jnp.concatenate([dst, dump])
    dst_h = dst_p.reshape(NC, NS, DNCH, DCHUNK)
    src_p = src_p.reshape(NS, NCH, CHUNK)
    dst_p = dst_p.reshape(NS, NCH, CHUNK)

    deg01 = _deg_call(dst_h)
    g = _mm_scale(x, W, deg01)
    scat = _scat_call(g[0], g[1], src_p, dst_p)
    return _epilogue(scat, g, deg01, b.reshape(1, D_FEAT),
                     W2.reshape(1, D_FEAT), b2.reshape(1, 1))


# speedup vs baseline: 1.0079x; 1.0079x over previous
import jax
import jax.numpy as jnp
from jax import lax
from jax.experimental import pallas as pl
from jax.experimental.pallas import tpu as pltpu
from jax.experimental.pallas import tpu_sc as plsc

N_NODES = 10000
D_FEAT = 128
N_EDGES = 320000

NC = 2
NS = 16
NW = NC * NS
DH = D_FEAT // NC
CHUNK = 128
N_PAD = 10240
ROWS_PER_TILE = N_PAD // NS
G_ROWS_PER_TILE = 624
NCH = 160
E_PAD = NS * NCH * CHUNK
NBUF = 2

_SC_MESH = dict(core_axis_name="c", subcore_axis_name="s")


def _deg_body(dst_hbm, out_hbm, didx_v, ones_v, zero_v, acc, sem):
    c = lax.axis_index("c")
    s = lax.axis_index("s")
    z16 = jnp.zeros((16,), jnp.float32)
    o16 = jnp.ones((16,), jnp.float32)

    def init_bufs(i, _):
        zero_v[pl.ds(i * 16, 16)] = z16
        return 0

    lax.fori_loop(0, ROWS_PER_TILE // 16, init_bufs, 0)
    for k in range(CHUNK // 16):
        ones_v[pl.ds(k * 16, 16)] = o16
    pltpu.sync_copy(dst_hbm.at[c, s], didx_v)
    pltpu.sync_copy(zero_v, acc.at[pl.ds(s * ROWS_PER_TILE, ROWS_PER_TILE)])
    plsc.subcore_barrier()

    def fire(j, _):
        pltpu.async_copy(ones_v, acc.at[didx_v.at[j]], sem, add=True)
        return 0

    lax.fori_loop(0, NCH // NC, fire, 0)

    def drain(j, _):
        pltpu.make_async_copy(ones_v, acc.at[didx_v.at[0]], sem).wait()
        return 0

    lax.fori_loop(0, NCH // NC, drain, 0)
    plsc.subcore_barrier()
    pltpu.sync_copy(acc.at[pl.ds(s * ROWS_PER_TILE, ROWS_PER_TILE)],
                    out_hbm.at[c, pl.ds(s * ROWS_PER_TILE, ROWS_PER_TILE)])


_deg_call = pl.kernel(
    _deg_body,
    out_type=jax.ShapeDtypeStruct((NC, N_PAD), jnp.float32),
    mesh=plsc.VectorSubcoreMesh(**_SC_MESH),
    scratch_types=[
        pltpu.VMEM((NCH // NC, CHUNK), jnp.int32),
        pltpu.VMEM((CHUNK,), jnp.float32),
        pltpu.VMEM((ROWS_PER_TILE,), jnp.float32),
        pltpu.VMEM_SHARED((N_PAD,), jnp.float32),
        pltpu.SemaphoreType.DMA,
    ],
)


def _scat_body(g0_hbm, g1_hbm, src_hbm, dst_hbm, out_hbm, sidx_v, didx_v,
               rows_v, gsp, acc, isem, dsem, gsem):
    c = lax.axis_index("c")
    s = lax.axis_index("s")
    z16 = jnp.zeros((16,), jnp.float32)

    @pl.when(c == 0)
    def _():
        pltpu.sync_copy(
            g0_hbm.at[pl.ds(s * G_ROWS_PER_TILE, G_ROWS_PER_TILE)],
            gsp.at[pl.ds(s * G_ROWS_PER_TILE, G_ROWS_PER_TILE)])

        @pl.when(s == 0)
        def _():
            tail = NS * G_ROWS_PER_TILE
            pltpu.sync_copy(g0_hbm.at[pl.ds(tail, N_NODES - tail)],
                            gsp.at[pl.ds(tail, N_NODES - tail)])

    @pl.when(c == 1)
    def _():
        pltpu.sync_copy(
            g1_hbm.at[pl.ds(s * G_ROWS_PER_TILE, G_ROWS_PER_TILE)],
            gsp.at[pl.ds(s * G_ROWS_PER_TILE, G_ROWS_PER_TILE)])

        @pl.when(s == 0)
        def _():
            tail = NS * G_ROWS_PER_TILE
            pltpu.sync_copy(g1_hbm.at[pl.ds(tail, N_NODES - tail)],
                            gsp.at[pl.ds(tail, N_NODES - tail)])

    def zero_rows(i, _):
        for k in range(DH // 16):
            rows_v[0, i, pl.ds(k * 16, 16)] = z16
        return 0

    lax.fori_loop(0, CHUNK, zero_rows, 0)
    for j in range(ROWS_PER_TILE // CHUNK):
        pltpu.sync_copy(
            rows_v.at[0], acc.at[pl.ds(s * ROWS_PER_TILE + j * CHUNK, CHUNK)])
    pltpu.async_copy(src_hbm.at[s, 0], sidx_v.at[0], isem)
    pltpu.async_copy(src_hbm.at[s, 1], sidx_v.at[1], isem)
    pltpu.async_copy(dst_hbm.at[s, 0], didx_v.at[0], dsem)
    pltpu.async_copy(dst_hbm.at[s, 1], didx_v.at[1], dsem)
    plsc.subcore_barrier()
    pltpu.make_async_copy(src_hbm.at[0, 0], sidx_v.at[0], isem).wait()
    pltpu.async_copy(gsp.at[sidx_v.at[0]], rows_v.at[0], gsem)

    def step(jg, _):
        for b in range(NBUF):
            j = jg * NBUF + b
            rows_b = rows_v.at[b]
            nb = (b + 1) % NBUF

            pltpu.make_async_copy(gsp.at[sidx_v.at[b]], rows_b, gsem).wait()

            @pl.when(j + 1 < NCH)
            def _():
                pltpu.make_async_copy(
                    src_hbm.at[0, 0], sidx_v.at[nb], isem).wait()
                pltpu.async_copy(gsp.at[sidx_v.at[nb]], rows_v.at[nb], gsem)

            @pl.when(j + NBUF < NCH)
            def _():
                pltpu.async_copy(src_hbm.at[s, j + NBUF], sidx_v.at[b], isem)

            pltpu.make_async_copy(dst_hbm.at[0, 0], didx_v.at[b], dsem).wait()
            pltpu.sync_copy(rows_b, acc.at[didx_v.at[b]], add=True)

            @pl.when(j + NBUF < NCH)
            def _():
                pltpu.async_copy(dst_hbm.at[s, j + NBUF], didx_v.at[b], dsem)

        return 0

    lax.fori_loop(0, NCH // NBUF, step, 0)
    plsc.subcore_barrier()
    pltpu.sync_copy(acc.at[pl.ds(s * ROWS_PER_TILE, ROWS_PER_TILE)],
                    out_hbm.at[c, pl.ds(s * ROWS_PER_TILE, ROWS_PER_TILE)])


_scat_call = pl.kernel(
    _scat_body,
    out_type=jax.ShapeDtypeStruct((NC, N_PAD, DH), jnp.float32),
    mesh=plsc.VectorSubcoreMesh(**_SC_MESH),
    compiler_params=pltpu.CompilerParams(use_tc_tiling_on_sc=False),
    scratch_types=[
        pltpu.VMEM((NBUF, CHUNK), jnp.int32),
        pltpu.VMEM((NBUF, CHUNK), jnp.int32),
        pltpu.VMEM((NBUF, CHUNK, DH), jnp.float32),
        pltpu.VMEM_SHARED((N_NODES, DH), jnp.float32),
        pltpu.VMEM_SHARED((N_PAD, DH), jnp.float32),
        pltpu.SemaphoreType.DMA,
        pltpu.SemaphoreType.DMA,
        pltpu.SemaphoreType.DMA,
    ],
)


_R = 1024


def _mm_scale_body(x_ref, w_ref, deg_ref, g_ref):
    h = jnp.dot(x_ref[...], w_ref[...], preferred_element_type=jnp.float32)
    deg = deg_ref[0, :] + deg_ref[1, :] + 1.0
    dinv = lax.rsqrt(deg)
    g = h * dinv[:, None]
    g_ref[0] = g[:, :DH]
    g_ref[1] = g[:, DH:]


def _mm_scale(x, W, deg01):
    return pl.pallas_call(
        _mm_scale_body,
        grid=(N_PAD // _R,),
        in_specs=[
            pl.BlockSpec((_R, D_FEAT), lambda i: (i, 0)),
            pl.BlockSpec((D_FEAT, D_FEAT), lambda i: (0, 0)),
            pl.BlockSpec((NC, _R), lambda i: (0, i)),
        ],
        out_specs=pl.BlockSpec((NC, _R, DH), lambda i: (0, i, 0)),
        out_shape=jax.ShapeDtypeStruct((NC, N_NODES, DH), jnp.float32),
    )(x, W, deg01)


def _epi_body(scat_ref, g_ref, deg_ref, b_ref, w2_ref, b2_ref, out_ref):
    deg = deg_ref[0, :] + deg_ref[1, :] + 1.0
    dinv = lax.rsqrt(deg)
    scat = jnp.concatenate([scat_ref[0], scat_ref[1]], axis=1)
    g = jnp.concatenate([g_ref[0], g_ref[1]], axis=1)
    agg = dinv[:, None] * (scat + g)
    z = jax.nn.relu(agg + b_ref[...])
    logits = jnp.sum(z * w2_ref[...], axis=1, keepdims=True) + b2_ref[...]
    out_ref[...] = jax.nn.sigmoid(logits)


def _epilogue(scat, g, deg01, b, W2, b2):
    return pl.pallas_call(
        _epi_body,
        grid=(N_PAD // _R,),
        in_specs=[
            pl.BlockSpec((NC, _R, DH), lambda i: (0, i, 0)),
            pl.BlockSpec((NC, _R, DH), lambda i: (0, i, 0)),
            pl.BlockSpec((NC, _R), lambda i: (0, i)),
            pl.BlockSpec((1, D_FEAT), lambda i: (0, 0)),
            pl.BlockSpec((1, D_FEAT), lambda i: (0, 0)),
            pl.BlockSpec((1, 1), lambda i: (0, 0)),
        ],
        out_specs=pl.BlockSpec((_R, 1), lambda i: (i, 0)),
        out_shape=jax.ShapeDtypeStruct((N_NODES, 1), jnp.float32),
    )(scat, g, deg01, b, W2, b2)


def kernel(x, edge_index, W, b, W2, b2):
    src = edge_index[0].astype(jnp.int32)
    dst = edge_index[1].astype(jnp.int32)
    npad = E_PAD - N_EDGES
    pad_src = jnp.arange(npad, dtype=jnp.int32) % N_NODES
    src_p = jnp.concatenate([src, pad_src])
    dump = N_NODES + jnp.arange(npad, dtype=jnp.int32) % (N_PAD - N_NODES)
    dst_p = jnp.concatenate([dst, dump])
    src_p = src_p.reshape(NS, NCH, CHUNK)
    dst_p = dst_p.reshape(NS, NCH, CHUNK)
    dst_h = dst_p.reshape(NC, NS, NCH // NC, CHUNK)

    deg01 = _deg_call(dst_h)
    g = _mm_scale(x, W, deg01)
    scat = _scat_call(g[0], g[1], src_p, dst_p)
    return _epilogue(scat, g, deg01, b.reshape(1, D_FEAT),
                     W2.reshape(1, D_FEAT), b2.reshape(1, 1))
